# baseline (speedup 1.0000x reference)
import functools

import jax
import jax.numpy as jnp
from jax.experimental import pallas as pl


def _agg_body(h_ref, nei_a, nei_b, wt_ref, b_ref, out_ref, *, inv_count, block):
    sa = jnp.sum(nei_a[...], axis=1) + h_ref[:block]
    sb = jnp.sum(nei_b[...], axis=1) + h_ref[block:]
    agg = jnp.concatenate([sa, sb], axis=0) * inv_count
    out_ref[...] = (
        jnp.dot(agg, wt_ref[...], preferred_element_type=jnp.float32) + b_ref[...]
    )


@jax.jit
def kernel(h, nei, W, b):
    n, in_feats = h.shape
    deg = nei.shape[1]
    out_feats = W.shape[0]

    block = 400
    grid = (pl.cdiv(n, 2 * block),)

    wt = W.T
    b2 = b.reshape(1, out_feats)

    body = functools.partial(_agg_body, inv_count=float(1.0 / (deg + 1)), block=block)

    return pl.pallas_call(
        body,
        grid=grid,
        in_specs=[
            pl.BlockSpec((2 * block, in_feats), lambda i: (i, 0)),
            pl.BlockSpec((block, deg, in_feats), lambda i: (2 * i, 0, 0)),
            pl.BlockSpec((block, deg, in_feats), lambda i: (2 * i + 1, 0, 0)),
            pl.BlockSpec((in_feats, out_feats), lambda i: (0, 0)),
            pl.BlockSpec((1, out_feats), lambda i: (0, 0)),
        ],
        out_specs=pl.BlockSpec((2 * block, out_feats), lambda i: (i, 0)),
        out_shape=jax.ShapeDtypeStruct((n, out_feats), jnp.float32),
    )(h, nei, nei, wt, b2)


# speedup vs baseline: 1.0562x; 1.0562x over previous
import functools

import jax
import jax.numpy as jnp
from jax.experimental import pallas as pl


def _agg_body(h_ref, nei_ref, wt_ref, b_ref, out_ref, *, inv_count):
    s = jnp.sum(nei_ref[...], axis=1) + h_ref[...]
    agg = s * inv_count
    out_ref[...] = (
        jnp.dot(agg, wt_ref[...], preferred_element_type=jnp.float32) + b_ref[...]
    )


@jax.jit
def kernel(h, nei, W, b):
    n, in_feats = h.shape
    deg = nei.shape[1]
    out_feats = W.shape[0]

    block = 448
    grid = (pl.cdiv(n, block),)

    wt = W.T
    b2 = b.reshape(1, out_feats)

    body = functools.partial(_agg_body, inv_count=float(1.0 / (deg + 1)))

    return pl.pallas_call(
        body,
        grid=grid,
        in_specs=[
            pl.BlockSpec((block, in_feats), lambda i: (i, 0)),
            pl.BlockSpec((block, deg, in_feats), lambda i: (i, 0, 0)),
            pl.BlockSpec((in_feats, out_feats), lambda i: (0, 0)),
            pl.BlockSpec((1, out_feats), lambda i: (0, 0)),
        ],
        out_specs=pl.BlockSpec((block, out_feats), lambda i: (i, 0)),
        out_shape=jax.ShapeDtypeStruct((n, out_feats), jnp.float32),
    )(h, nei, wt, b2)
